# baseline (device time: 52669 ns/iter reference)
import jax
import jax.numpy as jnp
from jax import lax
from jax.experimental import pallas as pl
from jax.experimental.pallas import tpu as pltpu

N_Z = 4
B, S, D = 2, 256, 1024
H, DH, DR = 16, 64, 32
DC = 64
BS = B * S
SCALE = (DH + DR) ** -0.5
BF16 = jnp.bfloat16
F32 = jnp.float32


def kernel(x, Wdkv, Wuk, Wuv, Wq, Wqr, Wkr, Wo):
    def body(x_ref, wdkv_ref, wuk_ref, wuv_ref, wq_ref, wqr_ref, wkr_ref,
             wo_ref, out_ref, comm_ref, o_ref, send_sems, recv_sems):
        mx = lax.axis_index("x")
        my = lax.axis_index("y")
        mz = lax.axis_index("z")

        def mm(a, b):
            return lax.dot_general(a, b, (((1,), (0,)), ((), ())),
                                   preferred_element_type=F32)

        def mm_t(a, b):
            return lax.dot_general(a, b, (((1,), (1,)), ((), ())),
                                   preferred_element_type=F32)

        wdkv_bf = wdkv_ref[...].astype(BF16)
        eye = (lax.broadcasted_iota(jnp.int32, (DC, DC), 0)
               == lax.broadcasted_iota(jnp.int32, (DC, DC), 1)).astype(BF16)
        comm_ref[mz, pl.ds(0, DC), :] = mm_t(eye, wdkv_bf).astype(BF16)
        comm_ref[mz, pl.ds(DC, DC), :] = wuk_ref[...].astype(BF16)
        comm_ref[mz, pl.ds(2 * DC, DC), :] = wuv_ref[...].astype(BF16)

        barrier = pltpu.get_barrier_semaphore()
        for dz in (1, 2, 3):
            pl.semaphore_signal(barrier, inc=1,
                                device_id=(mx, my, (mz + dz) % N_Z),
                                device_id_type=pl.DeviceIdType.MESH)
        pl.semaphore_wait(barrier, 3)

        sends = []
        for dz in (1, 2, 3):
            rdma = pltpu.make_async_remote_copy(
                src_ref=comm_ref.at[mz],
                dst_ref=comm_ref.at[mz],
                send_sem=send_sems.at[dz - 1],
                recv_sem=recv_sems.at[mz],
                device_id=(mx, my, (mz + dz) % N_Z),
                device_id_type=pl.DeviceIdType.MESH,
            )
            rdma.start()
            sends.append(rdma)

        x_bf = x_ref[...].astype(BF16).reshape(BS, D)
        q = mm(x_bf, wq_ref[...].astype(BF16)).astype(BF16)
        qr = mm(x_bf, wqr_ref[...].astype(BF16)).astype(BF16)
        kr = mm(x_bf, wkr_ref[...].astype(BF16)).astype(BF16)

        for dz in (1, 2, 3):
            sz = (mz + dz) % N_Z
            recv = pltpu.make_async_remote_copy(
                src_ref=comm_ref.at[sz],
                dst_ref=comm_ref.at[sz],
                send_sem=send_sems.at[dz - 1],
                recv_sem=recv_sems.at[sz],
                device_id=(mx, my, mz),
                device_id_type=pl.DeviceIdType.MESH,
            )
            recv.wait_recv()

        k_acc = jnp.zeros((BS, D), F32)
        v_acc = jnp.zeros((BS, D), F32)
        for z in range(N_Z):
            wd_t = comm_ref[z, pl.ds(0, DC), :]
            c_z = mm_t(x_bf, wd_t).astype(BF16)
            k_acc = k_acc + mm(c_z, comm_ref[z, pl.ds(DC, DC), :])
            v_acc = v_acc + mm(c_z, comm_ref[z, pl.ds(2 * DC, DC), :])
        k = k_acc.astype(BF16)
        v = v_acc.astype(BF16)

        for b in range(B):
            kr_b = kr[b * S:(b + 1) * S, :]
            for h in range(H):
                qh = q[b * S:(b + 1) * S, h * DH:(h + 1) * DH]
                kh = k[b * S:(b + 1) * S, h * DH:(h + 1) * DH]
                vh = v[b * S:(b + 1) * S, h * DH:(h + 1) * DH]
                qrh = qr[b * S:(b + 1) * S, h * DR:(h + 1) * DR]
                s_h = (mm_t(qh, kh) + mm_t(qrh, kr_b)) * SCALE
                m = jnp.max(s_h, axis=-1, keepdims=True)
                p = jnp.exp(s_h - m)
                p = p / jnp.sum(p, axis=-1, keepdims=True)
                o_ref[b * S:(b + 1) * S, h * DH:(h + 1) * DH] = (
                    mm(p.astype(BF16), vh).astype(BF16))

        out2d = mm(o_ref[...], wo_ref[...].astype(BF16))
        out_ref[0, :, :] = out2d[0:S, :]
        out_ref[1, :, :] = out2d[S:BS, :]

        for rdma in sends:
            rdma.wait_send()

    return pl.pallas_call(
        body,
        out_shape=jax.ShapeDtypeStruct((B, S, D), F32),
        in_specs=[pl.BlockSpec(memory_space=pltpu.VMEM)] * 8,
        out_specs=pl.BlockSpec(memory_space=pltpu.VMEM),
        scratch_shapes=[
            pltpu.VMEM((N_Z, 3 * DC, D), BF16),
            pltpu.VMEM((BS, D), BF16),
            pltpu.SemaphoreType.DMA((3,)),
            pltpu.SemaphoreType.DMA((N_Z,)),
        ],
        compiler_params=pltpu.CompilerParams(collective_id=0),
    )(x, Wdkv, Wuk, Wuv, Wq, Wqr, Wkr, Wo)
